# baseline (speedup 1.0000x reference)
import functools

import jax
import jax.numpy as jnp
from jax import lax
from jax.experimental import pallas as pl
from jax.experimental.pallas import tpu as pltpu
from jax.experimental.pallas import tpu_sc as plsc

D = 32
HIST = 50
NC, NS, L = 2, 16, 16
NW = NC * NS
B = 16384
BT = B // NW
C = 8
NCH = BT // C
CI = C * HIST
G = 16
NG = BT // G


def _worker_id():
    return lax.axis_index("s") * NC + lax.axis_index("c")


def _gather_body(uid_h, iid_h, uemb_h, iemb_h, ou_h, oi_h,
                 uid_v, iid_v, ublk, iblk, urows, irows, sem0, sem1):
    base = _worker_id() * BT
    pltpu.sync_copy(uid_h.at[pl.ds(base, BT)], uid_v)
    pltpu.sync_copy(iid_h.at[pl.ds(base, BT)], iid_v)

    sems = (sem0, sem1)

    def fire(g, k):
        uvec = uid_v[pl.ds(g * G, G)]
        ivec = iid_v[pl.ds(g * G, G)]
        for b in range(G):
            u8 = pl.multiple_of((uvec[b] >> 3) * 8, 8)
            i8 = pl.multiple_of((ivec[b] >> 3) * 8, 8)
            pltpu.async_copy(uemb_h.at[pl.ds(u8, 8)],
                             ublk.at[k, pl.ds(b * 8, 8)], sems[k])
            pltpu.async_copy(iemb_h.at[pl.ds(i8, 8)],
                             iblk.at[k, pl.ds(b * 8, 8)], sems[k])

    def drain(k):
        pltpu.make_async_copy(uemb_h.at[pl.ds(0, G * 8)], ublk.at[k], sems[k]).wait()
        pltpu.make_async_copy(iemb_h.at[pl.ds(0, G * 8)], iblk.at[k], sems[k]).wait()

    def extract(g, k):
        uvec = uid_v[pl.ds(g * G, G)]
        ivec = iid_v[pl.ds(g * G, G)]
        for b in range(G):
            ru = b * 8 + (uvec[b] & 7)
            ri = b * 8 + (ivec[b] & 7)
            for half in (0, 1):
                src = pl.ds(half * L, L)
                dst = pl.ds((g * G + b) * D + half * L, L)
                urows[dst] = ublk[k, ru, src]
                irows[dst] = iblk[k, ri, src]

    fire(0, 0)

    def step(t, carry):
        g0 = 2 * t
        fire(g0 + 1, 1)
        drain(0)
        extract(g0, 0)

        @pl.when(t < NG // 2 - 1)
        def _():
            fire(g0 + 2, 0)

        drain(1)
        extract(g0 + 1, 1)
        return carry

    lax.fori_loop(0, NG // 2, step, 0)
    pltpu.sync_copy(urows, ou_h.at[pl.ds(base * D, BT * D)])
    pltpu.sync_copy(irows, oi_h.at[pl.ds(base * D, BT * D)])


_gather = functools.partial(
    pl.kernel,
    out_type=(jax.ShapeDtypeStruct((B * D,), jnp.float32),
              jax.ShapeDtypeStruct((B * D,), jnp.float32)),
    mesh=plsc.VectorSubcoreMesh(core_axis_name="c", subcore_axis_name="s",
                                num_cores=NC, num_subcores=NS),
    scratch_types=[
        pltpu.VMEM((BT,), jnp.int32),
        pltpu.VMEM((BT,), jnp.int32),
        pltpu.VMEM((2, G * 8, D), jnp.float32),
        pltpu.VMEM((2, G * 8, D), jnp.float32),
        pltpu.VMEM((BT * D,), jnp.float32),
        pltpu.VMEM((BT * D,), jnp.float32),
        pltpu.SemaphoreType.DMA,
        pltpu.SemaphoreType.DMA,
    ],
    compiler_params=pltpu.CompilerParams(use_tc_tiling_on_sc=True),
)(_gather_body)


def _tmf_body(utag_h, itag_h, urows_h, irows_h, utemb_h, itemb_h,
              out_h,
              utag_i, itag_i, utr, itr, ur, ir, out_v,
              sem0, sem1):
    base = _worker_id() * BT

    pltpu.sync_copy(utag_h.at[pl.ds(base * HIST, BT * HIST)], utag_i)
    pltpu.sync_copy(itag_h.at[pl.ds(base * HIST, BT * HIST)], itag_i)

    def issue(g, k, sem):
        pltpu.async_copy(utemb_h.at[utag_i.at[pl.ds(g * CI, CI)]], utr.at[k], sem)
        pltpu.async_copy(itemb_h.at[itag_i.at[pl.ds(g * CI, CI)]], itr.at[k], sem)
        pltpu.async_copy(urows_h.at[pl.ds(base + g * C, C)], ur.at[k], sem)
        pltpu.async_copy(irows_h.at[pl.ds(base + g * C, C)], ir.at[k], sem)

    def drain(k, sem):
        pltpu.make_async_copy(utemb_h.at[pl.ds(0, CI)], utr.at[k], sem).wait()
        pltpu.make_async_copy(itemb_h.at[pl.ds(0, CI)], itr.at[k], sem).wait()
        pltpu.make_async_copy(urows_h.at[pl.ds(0, C)], ur.at[k], sem).wait()
        pltpu.make_async_copy(irows_h.at[pl.ds(0, C)], ir.at[k], sem).wait()

    def compute(g, k, init, lane_base):
        def body_b(b, acc):
            r0 = b * HIST
            u0 = utr[k, r0, pl.ds(0, L)]
            u1 = utr[k, r0, pl.ds(L, L)]
            i0 = itr[k, r0, pl.ds(0, L)]
            i1 = itr[k, r0, pl.ds(L, L)]
            for h in range(1, HIST):
                u0 = u0 + utr[k, r0 + h, pl.ds(0, L)]
                u1 = u1 + utr[k, r0 + h, pl.ds(L, L)]
                i0 = i0 + itr[k, r0 + h, pl.ds(0, L)]
                i1 = i1 + itr[k, r0 + h, pl.ds(L, L)]
            inv = 1.0 / HIST
            ru0 = ur[k, b, pl.ds(0, L)] + u0 * inv
            ru1 = ur[k, b, pl.ds(L, L)] + u1 * inv
            ri0 = ir[k, b, pl.ds(0, L)] + i0 * inv
            ri1 = ir[k, b, pl.ds(L, L)] + i1 * inv
            s = ru0 * ri0 + ru1 * ri1
            iot = lax.iota(jnp.int32, L)
            for k2 in (8, 4, 2, 1):
                s = s + s[jnp.bitwise_xor(iot, k2)]
            return jnp.where(iot == lane_base + b, s, acc)
        return lax.fori_loop(0, C, body_b, init)

    issue(0, 0, sem0)

    def pair(t, carry):
        g0 = 2 * t
        issue(g0 + 1, 1, sem1)
        drain(0, sem0)
        a = compute(g0, 0, jnp.zeros((L,), jnp.float32), 0)

        @pl.when(t < NCH // 2 - 1)
        def _():
            issue(g0 + 2, 0, sem0)

        drain(1, sem1)
        a = compute(g0 + 1, 1, a, C)
        out_v[pl.ds(t * 2 * C, 2 * C)] = a
        return carry

    lax.fori_loop(0, NCH // 2, pair, 0)

    pltpu.sync_copy(out_v, out_h.at[pl.ds(base, BT)])


_tmf = functools.partial(
    pl.kernel,
    out_type=jax.ShapeDtypeStruct((B,), jnp.float32),
    mesh=plsc.VectorSubcoreMesh(core_axis_name="c", subcore_axis_name="s",
                                num_cores=NC, num_subcores=NS),
    scratch_types=[
        pltpu.VMEM((BT * HIST,), jnp.int32),
        pltpu.VMEM((BT * HIST,), jnp.int32),
        pltpu.VMEM((2, CI, D), jnp.float32),
        pltpu.VMEM((2, CI, D), jnp.float32),
        pltpu.VMEM((2, C, D), jnp.float32),
        pltpu.VMEM((2, C, D), jnp.float32),
        pltpu.VMEM((BT,), jnp.float32),
        pltpu.SemaphoreType.DMA,
        pltpu.SemaphoreType.DMA,
    ],
    compiler_params=pltpu.CompilerParams(use_tc_tiling_on_sc=False),
)(_tmf_body)


def kernel(user_id, item_id, user_taggs, item_taggs,
           user_emb, item_emb, user_tagg_emb, item_tagg_emb):
    uid = user_id.astype(jnp.int32)
    iid = item_id.astype(jnp.int32)
    utag = user_taggs.astype(jnp.int32).reshape(-1)
    itag = item_taggs.astype(jnp.int32).reshape(-1)
    u_rows = jnp.take(user_emb, uid, axis=0)
    i_rows = jnp.take(item_emb, iid, axis=0)
    return _tmf(utag, itag, u_rows, i_rows, user_tagg_emb, item_tagg_emb)


# speedup vs baseline: 1.0836x; 1.0836x over previous
import functools

import jax
import jax.numpy as jnp
from jax import lax
from jax.experimental import pallas as pl
from jax.experimental.pallas import tpu as pltpu
from jax.experimental.pallas import tpu_sc as plsc

D = 32
HIST = 50
NC, NS, L = 2, 16, 16
NW = NC * NS
B = 16384
BT = B // NW
C = 8
NCH = BT // C
CI = C * HIST
G = 16
NG = BT // G


def _worker_id():
    return lax.axis_index("s") * NC + lax.axis_index("c")


def _gather_body(uid_h, iid_h, uemb_h, iemb_h, ou_h, oi_h,
                 uid_v, iid_v, ublk, iblk, urows, irows, sem0, sem1):
    base = _worker_id() * BT
    pltpu.sync_copy(uid_h.at[pl.ds(base, BT)], uid_v)
    pltpu.sync_copy(iid_h.at[pl.ds(base, BT)], iid_v)

    sems = (sem0, sem1)

    def fire(g, k):
        uvec = uid_v[pl.ds(g * G, G)]
        ivec = iid_v[pl.ds(g * G, G)]
        for b in range(G):
            u8 = pl.multiple_of((uvec[b] >> 3) * 8, 8)
            i8 = pl.multiple_of((ivec[b] >> 3) * 8, 8)
            pltpu.async_copy(uemb_h.at[pl.ds(u8, 8)],
                             ublk.at[k, pl.ds(b * 8, 8)], sems[k])
            pltpu.async_copy(iemb_h.at[pl.ds(i8, 8)],
                             iblk.at[k, pl.ds(b * 8, 8)], sems[k])

    def drain(k):
        pltpu.make_async_copy(uemb_h.at[pl.ds(0, G * 8)], ublk.at[k], sems[k]).wait()
        pltpu.make_async_copy(iemb_h.at[pl.ds(0, G * 8)], iblk.at[k], sems[k]).wait()

    def extract(g, k):
        uvec = uid_v[pl.ds(g * G, G)]
        ivec = iid_v[pl.ds(g * G, G)]
        for b in range(G):
            ru = b * 8 + (uvec[b] & 7)
            ri = b * 8 + (ivec[b] & 7)
            for half in (0, 1):
                src = pl.ds(half * L, L)
                dst = pl.ds((g * G + b) * D + half * L, L)
                urows[dst] = ublk[k, ru, src]
                irows[dst] = iblk[k, ri, src]

    fire(0, 0)

    def step(t, carry):
        g0 = 2 * t
        fire(g0 + 1, 1)
        drain(0)
        extract(g0, 0)

        @pl.when(t < NG // 2 - 1)
        def _():
            fire(g0 + 2, 0)

        drain(1)
        extract(g0 + 1, 1)
        return carry

    lax.fori_loop(0, NG // 2, step, 0)
    pltpu.sync_copy(urows, ou_h.at[pl.ds(base * D, BT * D)])
    pltpu.sync_copy(irows, oi_h.at[pl.ds(base * D, BT * D)])


_gather = functools.partial(
    pl.kernel,
    out_type=(jax.ShapeDtypeStruct((B * D,), jnp.float32),
              jax.ShapeDtypeStruct((B * D,), jnp.float32)),
    mesh=plsc.VectorSubcoreMesh(core_axis_name="c", subcore_axis_name="s",
                                num_cores=NC, num_subcores=NS),
    scratch_types=[
        pltpu.VMEM((BT,), jnp.int32),
        pltpu.VMEM((BT,), jnp.int32),
        pltpu.VMEM((2, G * 8, D), jnp.float32),
        pltpu.VMEM((2, G * 8, D), jnp.float32),
        pltpu.VMEM((BT * D,), jnp.float32),
        pltpu.VMEM((BT * D,), jnp.float32),
        pltpu.SemaphoreType.DMA,
        pltpu.SemaphoreType.DMA,
    ],
    compiler_params=pltpu.CompilerParams(use_tc_tiling_on_sc=True),
)(_gather_body)


def _tmf_body(utag_h, itag_h, urows_h, irows_h, utemb_h, itemb_h,
              out_h,
              utag_i, itag_i, utr, itr, ur, ir, out_v,
              sem0, sem1):
    base = _worker_id() * BT

    pltpu.sync_copy(utag_h.at[pl.ds(base * HIST, BT * HIST)], utag_i)
    pltpu.sync_copy(itag_h.at[pl.ds(base * HIST, BT * HIST)], itag_i)

    def issue(g, k, sem):
        pltpu.async_copy(utemb_h.at[utag_i.at[pl.ds(g * CI, CI)]], utr.at[k], sem)
        pltpu.async_copy(itemb_h.at[itag_i.at[pl.ds(g * CI, CI)]], itr.at[k], sem)
        pltpu.async_copy(urows_h.at[pl.ds((base + g * C) * D, C * D)], ur.at[k], sem)
        pltpu.async_copy(irows_h.at[pl.ds((base + g * C) * D, C * D)], ir.at[k], sem)

    def drain(k, sem):
        pltpu.make_async_copy(utemb_h.at[pl.ds(0, CI)], utr.at[k], sem).wait()
        pltpu.make_async_copy(itemb_h.at[pl.ds(0, CI)], itr.at[k], sem).wait()
        pltpu.make_async_copy(urows_h.at[pl.ds(0, C * D)], ur.at[k], sem).wait()
        pltpu.make_async_copy(irows_h.at[pl.ds(0, C * D)], ir.at[k], sem).wait()

    def compute(g, k, init, lane_base):
        himask = jnp.uint32(0xFFFF0000)

        def _lo(w):
            return plsc.bitcast(w << jnp.uint32(16), jnp.float32)

        def _hi(w):
            return plsc.bitcast(w & himask, jnp.float32)

        def body_b(b, acc):
            r0 = b * HIST
            wu = utr[k, r0, pl.ds(0, L)]
            wi = itr[k, r0, pl.ds(0, L)]
            u0, u1 = _lo(wu), _hi(wu)
            i0, i1 = _lo(wi), _hi(wi)
            for h in range(1, HIST):
                wu = utr[k, r0 + h, pl.ds(0, L)]
                wi = itr[k, r0 + h, pl.ds(0, L)]
                u0 = u0 + _lo(wu)
                u1 = u1 + _hi(wu)
                i0 = i0 + _lo(wi)
                i1 = i1 + _hi(wi)
            inv = 1.0 / HIST
            ru0 = ur[k, pl.ds(b * D, L)] + u0 * inv
            ru1 = ur[k, pl.ds(b * D + L, L)] + u1 * inv
            ri0 = ir[k, pl.ds(b * D, L)] + i0 * inv
            ri1 = ir[k, pl.ds(b * D + L, L)] + i1 * inv
            s = ru0 * ri0 + ru1 * ri1
            iot = lax.iota(jnp.int32, L)
            for k2 in (8, 4, 2, 1):
                s = s + s[jnp.bitwise_xor(iot, k2)]
            return jnp.where(iot == lane_base + b, s, acc)
        return lax.fori_loop(0, C, body_b, init)

    issue(0, 0, sem0)

    def pair(t, carry):
        g0 = 2 * t
        issue(g0 + 1, 1, sem1)
        drain(0, sem0)
        a = compute(g0, 0, jnp.zeros((L,), jnp.float32), 0)

        @pl.when(t < NCH // 2 - 1)
        def _():
            issue(g0 + 2, 0, sem0)

        drain(1, sem1)
        a = compute(g0 + 1, 1, a, C)
        out_v[pl.ds(t * 2 * C, 2 * C)] = a
        return carry

    lax.fori_loop(0, NCH // 2, pair, 0)

    pltpu.sync_copy(out_v, out_h.at[pl.ds(base, BT)])


_tmf = functools.partial(
    pl.kernel,
    out_type=jax.ShapeDtypeStruct((B,), jnp.float32),
    mesh=plsc.VectorSubcoreMesh(core_axis_name="c", subcore_axis_name="s",
                                num_cores=NC, num_subcores=NS),
    scratch_types=[
        pltpu.VMEM((BT * HIST,), jnp.int32),
        pltpu.VMEM((BT * HIST,), jnp.int32),
        pltpu.VMEM((2, CI, L), jnp.uint32),
        pltpu.VMEM((2, CI, L), jnp.uint32),
        pltpu.VMEM((2, C * D), jnp.float32),
        pltpu.VMEM((2, C * D), jnp.float32),
        pltpu.VMEM((BT,), jnp.float32),
        pltpu.SemaphoreType.DMA,
        pltpu.SemaphoreType.DMA,
    ],
    compiler_params=pltpu.CompilerParams(use_tc_tiling_on_sc=False,
                                         needs_layout_passes=False),
)(_tmf_body)


def _pack_bf16(t):
    t16 = t.astype(jnp.bfloat16)
    pairs = jnp.stack([t16[:, :L], t16[:, L:]], axis=-1)
    return jax.lax.bitcast_convert_type(pairs, jnp.uint32)


def kernel(user_id, item_id, user_taggs, item_taggs,
           user_emb, item_emb, user_tagg_emb, item_tagg_emb):
    uid = user_id.astype(jnp.int32)
    iid = item_id.astype(jnp.int32)
    utag = user_taggs.astype(jnp.int32).reshape(-1)
    itag = item_taggs.astype(jnp.int32).reshape(-1)
    utp = _pack_bf16(user_tagg_emb)
    itp = _pack_bf16(item_tagg_emb)
    u_rows = jnp.take(user_emb, uid, axis=0).reshape(-1)
    i_rows = jnp.take(item_emb, iid, axis=0).reshape(-1)
    return _tmf(utag, itag, u_rows, i_rows, utp, itp)


# speedup vs baseline: 1.2234x; 1.1290x over previous
import functools

import jax
import jax.numpy as jnp
from jax import lax
from jax.experimental import pallas as pl
from jax.experimental.pallas import tpu as pltpu
from jax.experimental.pallas import tpu_sc as plsc

D = 32
HIST = 50
HT = 2 * HIST
RW = 2 * D
NT = 100000
NC, NS, L = 2, 16, 16
NW = NC * NS
B = 16384
BT = B // NW
C = 16
NCH = BT // C
CI = C * HT


def _tmf_body(tag_h, rows_h, tbl_h, out_h,
              tag_i, tr, rw, out_v, sem0, sem1):
    wid = lax.axis_index("s") * NC + lax.axis_index("c")
    base = wid * BT

    pltpu.sync_copy(tag_h.at[pl.ds(base * HT, BT * HT)], tag_i)

    def issue(g, k, sem):
        pltpu.async_copy(tbl_h.at[tag_i.at[pl.ds(g * CI, CI)]], tr.at[k], sem)
        pltpu.async_copy(rows_h.at[pl.ds((base + g * C) * RW, C * RW)],
                         rw.at[k], sem)

    def drain(k, sem):
        pltpu.make_async_copy(tbl_h.at[pl.ds(0, CI)], tr.at[k], sem).wait()
        pltpu.make_async_copy(rows_h.at[pl.ds(0, C * RW)], rw.at[k], sem).wait()

    himask = jnp.uint32(0xFFFF0000)

    def _lo(w):
        return plsc.bitcast(w << jnp.uint32(16), jnp.float32)

    def _hi(w):
        return plsc.bitcast(w & himask, jnp.float32)

    def compute(g, k):
        def body_b(b, acc):
            r0 = b * HT
            wu = tr[k, r0, pl.ds(0, L)]
            wi = tr[k, r0 + HIST, pl.ds(0, L)]
            u0, u1 = _lo(wu), _hi(wu)
            i0, i1 = _lo(wi), _hi(wi)
            for h in range(1, HIST):
                wu = tr[k, r0 + h, pl.ds(0, L)]
                wi = tr[k, r0 + HIST + h, pl.ds(0, L)]
                u0 = u0 + _lo(wu)
                u1 = u1 + _hi(wu)
                i0 = i0 + _lo(wi)
                i1 = i1 + _hi(wi)
            inv = 1.0 / HIST
            ru0 = rw[k, pl.ds(b * RW, L)] + u0 * inv
            ru1 = rw[k, pl.ds(b * RW + L, L)] + u1 * inv
            ri0 = rw[k, pl.ds(b * RW + D, L)] + i0 * inv
            ri1 = rw[k, pl.ds(b * RW + D + L, L)] + i1 * inv
            s = ru0 * ri0 + ru1 * ri1
            iot = lax.iota(jnp.int32, L)
            for k2 in (8, 4, 2, 1):
                s = s + s[jnp.bitwise_xor(iot, k2)]
            return jnp.where(iot == b, s, acc)
        a = lax.fori_loop(0, C, body_b, jnp.zeros((L,), jnp.float32))
        out_v[pl.ds(g * C, C)] = a

    issue(0, 0, sem0)

    def pair(t, carry):
        g0 = 2 * t
        issue(g0 + 1, 1, sem1)
        drain(0, sem0)
        compute(g0, 0)

        @pl.when(t < NCH // 2 - 1)
        def _():
            issue(g0 + 2, 0, sem0)

        drain(1, sem1)
        compute(g0 + 1, 1)
        return carry

    lax.fori_loop(0, NCH // 2, pair, 0)

    pltpu.sync_copy(out_v, out_h.at[pl.ds(base, BT)])


_tmf = functools.partial(
    pl.kernel,
    out_type=jax.ShapeDtypeStruct((B,), jnp.float32),
    mesh=plsc.VectorSubcoreMesh(core_axis_name="c", subcore_axis_name="s",
                                num_cores=NC, num_subcores=NS),
    scratch_types=[
        pltpu.VMEM((BT * HT,), jnp.int32),
        pltpu.VMEM((2, CI, L), jnp.uint32),
        pltpu.VMEM((2, C * RW), jnp.float32),
        pltpu.VMEM((BT,), jnp.float32),
        pltpu.SemaphoreType.DMA,
        pltpu.SemaphoreType.DMA,
    ],
    compiler_params=pltpu.CompilerParams(use_tc_tiling_on_sc=False,
                                         needs_layout_passes=False),
)(_tmf_body)


def _pack_bf16(t):
    t16 = t.astype(jnp.bfloat16)
    pairs = jnp.stack([t16[:, :L], t16[:, L:]], axis=-1)
    return jax.lax.bitcast_convert_type(pairs, jnp.uint32)


def kernel(user_id, item_id, user_taggs, item_taggs,
           user_emb, item_emb, user_tagg_emb, item_tagg_emb):
    uid = user_id.astype(jnp.int32)
    iid = item_id.astype(jnp.int32)
    tag_idx = jnp.concatenate(
        [user_taggs.astype(jnp.int32), item_taggs.astype(jnp.int32) + NT],
        axis=1).reshape(-1)
    tbl = _pack_bf16(jnp.concatenate([user_tagg_emb, item_tagg_emb], axis=0))
    rows = jnp.concatenate(
        [jnp.take(user_emb, uid, axis=0), jnp.take(item_emb, iid, axis=0)],
        axis=1).reshape(-1)
    return _tmf(tag_idx, rows, tbl)
